# baseline (speedup 1.0000x reference)
import functools

import jax
import jax.numpy as jnp
from jax import lax
from jax.experimental import pallas as pl
from jax.experimental.pallas import tpu as pltpu
from jax.experimental.pallas import tpu_sc as plsc

_B, _N, _K, _HID = 64, 1024, 10, 16
_NC, _NS, _L = 2, 16, 16
_NW = _NC * _NS
_BPW = _B // _NW


def _tc_body(cap_ref, w1_ref, b1_ref, w2_ref, b2_ref,
             xs_ref, ys_ref, dm_ref,
             c_ref, s_ref, nd_ref, dd_ref):
    x = xs_ref[...]
    y = ys_ref[...]
    dm = dm_ref[...]
    cap = cap_ref[0, 0]

    x0 = xs_ref[:, 0:1]
    y0 = ys_ref[:, 0:1]
    dx0 = x - x0
    dy0 = y - y0
    dd = jnp.sqrt(dx0 * dx0 + dy0 * dy0 + jnp.float32(1e-12))
    nd = dm / cap
    lane = lax.broadcasted_iota(jnp.int32, (8, _N), 1)
    isd = jnp.where(lane == 0, jnp.float32(1.0), jnp.float32(0.0))
    feats = (x, y, nd, dd, isd)

    nd_ref[...] = nd
    dd_ref[...] = dd

    theta = jnp.zeros((8, _N), jnp.float32)
    for j in range(_HID):
        hj = feats[0] * w1_ref[0, j]
        for d in range(1, 5):
            hj = hj + feats[d] * w1_ref[d, j]
        hj = hj + b1_ref[0, j]
        theta = theta + jnp.tanh(hj) * w2_ref[j, 0]
    theta = theta + b2_ref[0, 0]
    c_ref[...] = jnp.cos(theta)
    s_ref[...] = jnp.sin(theta)


def _tc_rot_body(p0_ref, p1_ref, c_ref, s_ref, psix_ref, psiy_ref):
    p0 = p0_ref[...]
    p1 = p1_ref[...]
    c = c_ref[...]
    s = s_ref[...]
    norm = jnp.sqrt(p0 * p0 + p1 * p1) + jnp.float32(1e-8)
    px = p0 / norm
    py = p1 / norm
    psix_ref[...] = c * px - s * py
    psiy_ref[...] = s * px + c * py


@jax.jit
def _run_tc(xs, ys, demands, capacity, W1, b1, W2, b2):
    smem = pl.BlockSpec(memory_space=pltpu.SMEM)
    row = pl.BlockSpec((8, _N), lambda b: (b, 0))
    return pl.pallas_call(
        _tc_body,
        grid=(_B // 8,),
        in_specs=[smem] * 5 + [row, row, row],
        out_specs=[row, row, row, row],
        out_shape=[jax.ShapeDtypeStruct((_B, _N), jnp.float32)] * 4,
    )(capacity.reshape(1, 1), W1,
      b1.reshape(1, _HID), W2, b2.reshape(1, 1), xs, ys, demands)


@jax.jit
def _run_rot(p0, p1, c, s):
    row = pl.BlockSpec((8, _N), lambda b: (b, 0))
    return pl.pallas_call(
        _tc_rot_body,
        grid=(_B // 8,),
        in_specs=[row] * 4,
        out_specs=[row, row],
        out_shape=[jax.ShapeDtypeStruct((_B, _N), jnp.float32)] * 2,
    )(p0, p1, c, s)


_UNR = 4


def _sc_knn_body(xs_hbm, ys_hbm, out_hbm, xv, yv, ov, dbuf, jbuf, d2c):
    wid = lax.axis_index("s") * _NC + lax.axis_index("c")
    iota = lax.iota(jnp.int32, _L)
    zero_i = jnp.zeros((_L,), jnp.int32)
    big = jnp.full((_L,), 3.4e38, jnp.float32)
    lane16 = iota * _L
    for bi in range(_BPW):
        b = wid * _BPW + bi
        pltpu.sync_copy(xs_hbm.at[b], xv)
        pltpu.sync_copy(ys_hbm.at[b], yv)

        def group_body(g, _):
            qi = iota + g * _L
            qx = plsc.load_gather(xv, [qi])
            qy = plsc.load_gather(yv, [qi])
            splats = [zero_i + u for u in range(_L)]

            def chunk_d2(t, u, cxv, cyv, selfg):
                cx = cxv.at[splats[u]].get(mode="promise_in_bounds")
                cy = cyv.at[splats[u]].get(mode="promise_in_bounds")
                dxv = qx - cx
                dyv = qy - cy
                d2 = dxv * dxv + dyv * dyv
                return jnp.where(selfg & (iota == u), jnp.float32(1e9), d2)

            def p1_body(t, lads):
                cvec = iota + t * _L
                cxv = plsc.load_gather(xv, [cvec])
                cyv = plsc.load_gather(yv, [cvec])
                selfg = jnp.broadcast_to(t == g, (_L,))
                cbase = t * (_L * _L)
                lads = list(lads)
                for u in range(_L):
                    d2 = chunk_d2(t, u, cxv, cyv, selfg)
                    plsc.store_scatter(d2c, [iota + (cbase + u * _L)], d2)
                    lad = lads[u % _UNR]
                    nl = [jnp.minimum(lad[0], d2)]
                    for r in range(1, _K):
                        nl.append(jnp.minimum(
                            lad[r], jnp.maximum(d2, lad[r - 1])))
                    lads[u % _UNR] = tuple(nl)
                return tuple(lads)

            lads = lax.fori_loop(0, _N // _L, p1_body,
                                 ((big,) * _K,) * _UNR)

            def merge10(a, bb):
                out = []
                for r in range(_K):
                    m = jnp.minimum(a[r], bb[r])
                    for s in range(1, r + 1):
                        m = jnp.minimum(m, jnp.maximum(a[s - 1], bb[r - s]))
                    out.append(m)
                return tuple(out)

            m01 = merge10(lads[0], lads[1])
            m23 = merge10(lads[2], lads[3])
            b9 = jnp.minimum(m01[_K - 1], m23[_K - 1])
            for s in range(1, _K):
                b9 = jnp.minimum(b9, jnp.maximum(m01[s - 1], m23[_K - 1 - s]))

            def p2_body(t, cur):
                cbase = t * (_L * _L)
                base = zero_i + t * _L
                for u in range(_L):
                    d2 = plsc.load_gather(d2c, [iota + (cbase + u * _L)])
                    m = d2 <= b9
                    pos = lane16 + jnp.minimum(cur, _L - 1)
                    plsc.store_scatter(jbuf, [pos], base + u, mask=m)
                    plsc.store_scatter(dbuf, [pos], d2, mask=m)
                    cur = cur + jnp.where(m, 1, 0)
                return cur

            cur = lax.fori_loop(0, _N // _L, p2_body, zero_i)

            for q in range(_L):
                cnt = cur[q]
                keys = dbuf[pl.ds(q * _L, _L)]
                vals = jbuf[pl.ds(q * _L, _L)]
                keys = jnp.where(iota < cnt, keys, jnp.float32(3.4e38))
                _, sv = plsc.sort_key_val(keys, vals)
                outv = (g * _L + q) * _K + iota
                plsc.store_scatter(ov, [outv], sv, mask=iota < _K)
            return 0

        lax.fori_loop(0, _N // _L, group_body, 0)
        pltpu.sync_copy(ov, out_hbm.at[b])


_sc_mesh = plsc.VectorSubcoreMesh(core_axis_name="c", subcore_axis_name="s",
                                  num_cores=_NC, num_subcores=_NS)

_sc_knn = functools.partial(
    pl.kernel,
    out_type=jax.ShapeDtypeStruct((_B, _N * _K), jnp.int32),
    mesh=_sc_mesh,
    compiler_params=pltpu.CompilerParams(needs_layout_passes=False),
    scratch_types=[
        pltpu.VMEM((_N,), jnp.float32),
        pltpu.VMEM((_N,), jnp.float32),
        pltpu.VMEM((_N * _K,), jnp.int32),
        pltpu.VMEM((_L * _L,), jnp.float32),
        pltpu.VMEM((_L * _L,), jnp.int32),
        pltpu.VMEM((_N * _L,), jnp.float32),
    ],
)(_sc_knn_body)


def kernel(coords, demands, capacity, W_amp, b_amp, W1, b1, W2, b2):
    xs = coords[:, :, 0]
    ys = coords[:, :, 1]
    c, s, nd, dd = _run_tc(xs, ys, demands, capacity, W1, b1, W2, b2)
    knn_flat = _sc_knn(xs, ys)
    isd = jnp.zeros((_B, _N), jnp.float32).at[:, 0].set(1.0)
    features = jnp.stack([xs, ys, nd, dd, isd], axis=-1)
    p = features @ W_amp + b_amp
    psix, psiy = _run_rot(p[..., 0], p[..., 1], c, s)
    psi_prime = jnp.stack([psix, psiy], axis=-1)
    knn = knn_flat.reshape(_B, _N, _K)
    return psi_prime, features, knn


# speedup vs baseline: 1.7461x; 1.7461x over previous
import functools

import jax
import jax.numpy as jnp
from jax import lax
from jax.experimental import pallas as pl
from jax.experimental.pallas import tpu as pltpu
from jax.experimental.pallas import tpu_sc as plsc

_B, _N, _K, _HID = 64, 1024, 10, 16
_NC, _NS, _L = 2, 16, 16
_NW = _NC * _NS
_BPW = _B // _NW


def _tc_body(cap_ref, w1_ref, b1_ref, w2_ref, b2_ref,
             xs_ref, ys_ref, dm_ref,
             c_ref, s_ref, nd_ref, dd_ref):
    x = xs_ref[...]
    y = ys_ref[...]
    dm = dm_ref[...]
    cap = cap_ref[0, 0]

    x0 = xs_ref[:, 0:1]
    y0 = ys_ref[:, 0:1]
    dx0 = x - x0
    dy0 = y - y0
    dd = jnp.sqrt(dx0 * dx0 + dy0 * dy0 + jnp.float32(1e-12))
    nd = dm / cap
    lane = lax.broadcasted_iota(jnp.int32, (8, _N), 1)
    isd = jnp.where(lane == 0, jnp.float32(1.0), jnp.float32(0.0))
    feats = (x, y, nd, dd, isd)

    nd_ref[...] = nd
    dd_ref[...] = dd

    theta = jnp.zeros((8, _N), jnp.float32)
    for j in range(_HID):
        hj = feats[0] * w1_ref[0, j]
        for d in range(1, 5):
            hj = hj + feats[d] * w1_ref[d, j]
        hj = hj + b1_ref[0, j]
        theta = theta + jnp.tanh(hj) * w2_ref[j, 0]
    theta = theta + b2_ref[0, 0]
    c_ref[...] = jnp.cos(theta)
    s_ref[...] = jnp.sin(theta)


def _tc_rot_body(p0_ref, p1_ref, c_ref, s_ref, psix_ref, psiy_ref):
    p0 = p0_ref[...]
    p1 = p1_ref[...]
    c = c_ref[...]
    s = s_ref[...]
    norm = jnp.sqrt(p0 * p0 + p1 * p1) + jnp.float32(1e-8)
    px = p0 / norm
    py = p1 / norm
    psix_ref[...] = c * px - s * py
    psiy_ref[...] = s * px + c * py


@jax.jit
def _run_tc(xs, ys, demands, capacity, W1, b1, W2, b2):
    smem = pl.BlockSpec(memory_space=pltpu.SMEM)
    row = pl.BlockSpec((8, _N), lambda b: (b, 0))
    return pl.pallas_call(
        _tc_body,
        grid=(_B // 8,),
        in_specs=[smem] * 5 + [row, row, row],
        out_specs=[row, row, row, row],
        out_shape=[jax.ShapeDtypeStruct((_B, _N), jnp.float32)] * 4,
    )(capacity.reshape(1, 1), W1,
      b1.reshape(1, _HID), W2, b2.reshape(1, 1), xs, ys, demands)


@jax.jit
def _run_rot(p0, p1, c, s):
    row = pl.BlockSpec((8, _N), lambda b: (b, 0))
    return pl.pallas_call(
        _tc_rot_body,
        grid=(_B // 8,),
        in_specs=[row] * 4,
        out_specs=[row, row],
        out_shape=[jax.ShapeDtypeStruct((_B, _N), jnp.float32)] * 2,
    )(p0, p1, c, s)


_UNR = 4


def _sc_knn_body(xs_hbm, ys_hbm, out_hbm, xv, yv, ov, jbuf):
    wid = lax.axis_index("s") * _NC + lax.axis_index("c")
    iota = lax.iota(jnp.int32, _L)
    zero_i = jnp.zeros((_L,), jnp.int32)
    big = jnp.full((_L,), 3.4e38, jnp.float32)
    lane16 = iota * _L
    for bi in range(_BPW):
        b = wid * _BPW + bi
        pltpu.sync_copy(xs_hbm.at[b], xv)
        pltpu.sync_copy(ys_hbm.at[b], yv)

        def group_body(g, _):
            qi = iota + g * _L
            qx = plsc.load_gather(xv, [qi])
            qy = plsc.load_gather(yv, [qi])
            splats = [zero_i + u for u in range(_L)]

            def chunk_d2(t, u, cxv, cyv, selfg):
                cx = cxv.at[splats[u]].get(mode="promise_in_bounds")
                cy = cyv.at[splats[u]].get(mode="promise_in_bounds")
                dxv = qx - cx
                dyv = qy - cy
                d2 = dxv * dxv + dyv * dyv
                return jnp.where(selfg & (iota == u), jnp.float32(1e9), d2)

            def p1_body(t, lads):
                cvec = iota + t * _L
                cxv = plsc.load_gather(xv, [cvec])
                cyv = plsc.load_gather(yv, [cvec])
                selfg = jnp.broadcast_to(t == g, (_L,))
                lads = list(lads)
                for u in range(_L):
                    d2 = chunk_d2(t, u, cxv, cyv, selfg)
                    lad = lads[u % _UNR]
                    nl = [jnp.minimum(lad[0], d2)]
                    for r in range(1, _K):
                        nl.append(jnp.minimum(
                            lad[r], jnp.maximum(d2, lad[r - 1])))
                    lads[u % _UNR] = tuple(nl)
                return tuple(lads)

            lads = lax.fori_loop(0, _N // _L, p1_body,
                                 ((big,) * _K,) * _UNR)

            def merge10(a, bb):
                out = []
                for r in range(_K):
                    m = jnp.minimum(a[r], bb[r])
                    for s in range(1, r + 1):
                        m = jnp.minimum(m, jnp.maximum(a[s - 1], bb[r - s]))
                    out.append(m)
                return tuple(out)

            m01 = merge10(lads[0], lads[1])
            m23 = merge10(lads[2], lads[3])
            b9 = jnp.minimum(m01[_K - 1], m23[_K - 1])
            for s in range(1, _K):
                b9 = jnp.minimum(b9, jnp.maximum(m01[s - 1], m23[_K - 1 - s]))

            def p2_body(t, cur):
                cvec = iota + t * _L
                cxv = plsc.load_gather(xv, [cvec])
                cyv = plsc.load_gather(yv, [cvec])
                selfg = jnp.broadcast_to(t == g, (_L,))
                base = zero_i + t * _L
                for u in range(_L):
                    d2 = chunk_d2(t, u, cxv, cyv, selfg)
                    m = d2 <= b9
                    pos = lane16 + jnp.minimum(cur, _L - 1)
                    plsc.store_scatter(jbuf, [pos], base + u, mask=m)
                    cur = cur + jnp.where(m, 1, 0)
                return cur

            cur = lax.fori_loop(0, _N // _L, p2_body, zero_i)

            for q in range(_L):
                cnt = cur[q]
                vals = jbuf[pl.ds(q * _L, _L)]
                vc = jnp.clip(vals, 0, _N - 1)
                cxs = plsc.load_gather(xv, [vc])
                cys = plsc.load_gather(yv, [vc])
                qxs = qx.at[splats[q]].get(mode="promise_in_bounds")
                qys = qy.at[splats[q]].get(mode="promise_in_bounds")
                dxv = qxs - cxs
                dyv = qys - cys
                keys = dxv * dxv + dyv * dyv
                keys = jnp.where(iota < cnt, keys, jnp.float32(3.4e38))
                _, sv = plsc.sort_key_val(keys, vals)
                outv = (g * _L + q) * _K + iota
                plsc.store_scatter(ov, [outv], sv, mask=iota < _K)
            return 0

        lax.fori_loop(0, _N // _L, group_body, 0)
        pltpu.sync_copy(ov, out_hbm.at[b])


_sc_mesh = plsc.VectorSubcoreMesh(core_axis_name="c", subcore_axis_name="s",
                                  num_cores=_NC, num_subcores=_NS)

_sc_knn = functools.partial(
    pl.kernel,
    out_type=jax.ShapeDtypeStruct((_B, _N * _K), jnp.int32),
    mesh=_sc_mesh,
    compiler_params=pltpu.CompilerParams(needs_layout_passes=False),
    scratch_types=[
        pltpu.VMEM((_N,), jnp.float32),
        pltpu.VMEM((_N,), jnp.float32),
        pltpu.VMEM((_N * _K,), jnp.int32),
        pltpu.VMEM((_L * _L,), jnp.int32),
    ],
)(_sc_knn_body)


def kernel(coords, demands, capacity, W_amp, b_amp, W1, b1, W2, b2):
    xs = coords[:, :, 0]
    ys = coords[:, :, 1]
    c, s, nd, dd = _run_tc(xs, ys, demands, capacity, W1, b1, W2, b2)
    knn_flat = _sc_knn(xs, ys)
    isd = jnp.zeros((_B, _N), jnp.float32).at[:, 0].set(1.0)
    features = jnp.stack([xs, ys, nd, dd, isd], axis=-1)
    p = features @ W_amp + b_amp
    psix, psiy = _run_rot(p[..., 0], p[..., 1], c, s)
    psi_prime = jnp.stack([psix, psiy], axis=-1)
    knn = knn_flat.reshape(_B, _N, _K)
    return psi_prime, features, knn


# speedup vs baseline: 2.6084x; 1.4938x over previous
import functools

import jax
import jax.numpy as jnp
from jax import lax
from jax.experimental import pallas as pl
from jax.experimental.pallas import tpu as pltpu
from jax.experimental.pallas import tpu_sc as plsc

_B, _N, _K, _HID = 64, 1024, 10, 16
_NC, _NS, _L = 2, 16, 16
_NW = _NC * _NS
_BPW = _B // _NW


def _tc_body(cap_ref, w1_ref, b1_ref, w2_ref, b2_ref,
             xs_ref, ys_ref, dm_ref,
             c_ref, s_ref, nd_ref, dd_ref):
    x = xs_ref[...]
    y = ys_ref[...]
    dm = dm_ref[...]
    cap = cap_ref[0, 0]

    x0 = xs_ref[:, 0:1]
    y0 = ys_ref[:, 0:1]
    dx0 = x - x0
    dy0 = y - y0
    dd = jnp.sqrt(dx0 * dx0 + dy0 * dy0 + jnp.float32(1e-12))
    nd = dm / cap
    lane = lax.broadcasted_iota(jnp.int32, (8, _N), 1)
    isd = jnp.where(lane == 0, jnp.float32(1.0), jnp.float32(0.0))
    feats = (x, y, nd, dd, isd)

    nd_ref[...] = nd
    dd_ref[...] = dd

    theta = jnp.zeros((8, _N), jnp.float32)
    for j in range(_HID):
        hj = feats[0] * w1_ref[0, j]
        for d in range(1, 5):
            hj = hj + feats[d] * w1_ref[d, j]
        hj = hj + b1_ref[0, j]
        theta = theta + jnp.tanh(hj) * w2_ref[j, 0]
    theta = theta + b2_ref[0, 0]
    c_ref[...] = jnp.cos(theta)
    s_ref[...] = jnp.sin(theta)


def _tc_rot_body(p0_ref, p1_ref, c_ref, s_ref, psix_ref, psiy_ref):
    p0 = p0_ref[...]
    p1 = p1_ref[...]
    c = c_ref[...]
    s = s_ref[...]
    norm = jnp.sqrt(p0 * p0 + p1 * p1) + jnp.float32(1e-8)
    px = p0 / norm
    py = p1 / norm
    psix_ref[...] = c * px - s * py
    psiy_ref[...] = s * px + c * py


@jax.jit
def _run_tc(xs, ys, demands, capacity, W1, b1, W2, b2):
    smem = pl.BlockSpec(memory_space=pltpu.SMEM)
    row = pl.BlockSpec((8, _N), lambda b: (b, 0))
    return pl.pallas_call(
        _tc_body,
        grid=(_B // 8,),
        in_specs=[smem] * 5 + [row, row, row],
        out_specs=[row, row, row, row],
        out_shape=[jax.ShapeDtypeStruct((_B, _N), jnp.float32)] * 4,
    )(capacity.reshape(1, 1), W1,
      b1.reshape(1, _HID), W2, b2.reshape(1, 1), xs, ys, demands)


@jax.jit
def _run_rot(p0, p1, c, s):
    row = pl.BlockSpec((8, _N), lambda b: (b, 0))
    return pl.pallas_call(
        _tc_rot_body,
        grid=(_B // 8,),
        in_specs=[row] * 4,
        out_specs=[row, row],
        out_shape=[jax.ShapeDtypeStruct((_B, _N), jnp.float32)] * 2,
    )(p0, p1, c, s)


_CAP = 32


def _sc_knn_body(xs_hbm, ys_hbm, out_hbm, xv, yv, ov, jbuf):
    wid = lax.axis_index("s") * _NC + lax.axis_index("c")
    iota = lax.iota(jnp.int32, _L)
    zero_i = jnp.zeros((_L,), jnp.int32)
    big = jnp.full((_L,), 3.4e38, jnp.float32)
    lane_cap = iota * _CAP
    for bi in range(_BPW):
        b = wid * _BPW + bi
        pltpu.sync_copy(xs_hbm.at[b], xv)
        pltpu.sync_copy(ys_hbm.at[b], yv)

        def group_body(g, _):
            qi = iota + g * _L
            qx = plsc.load_gather(xv, [qi])
            qy = plsc.load_gather(yv, [qi])
            splats = [zero_i + u for u in range(_L)]

            def chunk_d2(t, u, cxv, cyv, selfg):
                cx = cxv.at[splats[u]].get(mode="promise_in_bounds")
                cy = cyv.at[splats[u]].get(mode="promise_in_bounds")
                dxv = qx - cx
                dyv = qy - cy
                d2 = dxv * dxv + dyv * dyv
                return jnp.where(selfg & (iota == u), jnp.float32(1e9), d2)

            def p1_body(t, mins):
                cvec = iota + t * _L
                cxv = plsc.load_gather(xv, [cvec])
                cyv = plsc.load_gather(yv, [cvec])
                selfg = jnp.broadcast_to(t == g, (_L,))
                return tuple(
                    jnp.minimum(mins[u], chunk_d2(t, u, cxv, cyv, selfg))
                    for u in range(_L))

            mins = list(lax.fori_loop(0, _N // _L, p1_body, (big,) * _L))

            for k in (2, 4, 8, 16):
                j = k // 2
                while j >= 1:
                    for i in range(_L):
                        l2 = i ^ j
                        if l2 > i:
                            lo = jnp.minimum(mins[i], mins[l2])
                            hi = jnp.maximum(mins[i], mins[l2])
                            if (i & k) == 0:
                                mins[i], mins[l2] = lo, hi
                            else:
                                mins[i], mins[l2] = hi, lo
                    j //= 2
            b9 = mins[_K - 1]

            def p2_body(t, cur):
                cvec = iota + t * _L
                cxv = plsc.load_gather(xv, [cvec])
                cyv = plsc.load_gather(yv, [cvec])
                selfg = jnp.broadcast_to(t == g, (_L,))
                base = zero_i + t * _L
                for u in range(_L):
                    d2 = chunk_d2(t, u, cxv, cyv, selfg)
                    m = d2 <= b9
                    pos = lane_cap + jnp.minimum(cur, _CAP - 1)
                    plsc.store_scatter(jbuf, [pos], base + u, mask=m)
                    cur = cur + jnp.where(m, 1, 0)
                return cur

            cur = lax.fori_loop(0, _N // _L, p2_body, zero_i)

            for q in range(_L):
                cnt = cur[q]
                qxs = qx.at[splats[q]].get(mode="promise_in_bounds")
                qys = qy.at[splats[q]].get(mode="promise_in_bounds")

                def seg(off, _q=q, _cnt=cnt, _qxs=qxs, _qys=qys):
                    vals = jbuf[pl.ds(_q * _CAP + off, _L)]
                    vc = jnp.clip(vals, 0, _N - 1)
                    cxs = plsc.load_gather(xv, [vc])
                    cys = plsc.load_gather(yv, [vc])
                    dxv = _qxs - cxs
                    dyv = _qys - cys
                    keys = dxv * dxv + dyv * dyv
                    keys = jnp.where(iota + off < _cnt, keys,
                                     jnp.float32(3.4e38))
                    return keys, vals

                ka, va = seg(0)
                kb, vb = seg(_L)
                ska, sva = plsc.sort_key_val(ka, va)
                skb, svb = plsc.sort_key_val(kb, vb)
                rkb = lax.rev(skb, (0,))
                rvb = lax.rev(svb, (0,))
                mm = ska <= rkb
                mk = jnp.where(mm, ska, rkb)
                mv = jnp.where(mm, sva, rvb)
                _, sv = plsc.sort_key_val(mk, mv)
                outv = (g * _L + q) * _K + iota
                plsc.store_scatter(ov, [outv], sv, mask=iota < _K)
            return 0

        lax.fori_loop(0, _N // _L, group_body, 0)
        pltpu.sync_copy(ov, out_hbm.at[b])


_sc_mesh = plsc.VectorSubcoreMesh(core_axis_name="c", subcore_axis_name="s",
                                  num_cores=_NC, num_subcores=_NS)

_sc_knn = functools.partial(
    pl.kernel,
    out_type=jax.ShapeDtypeStruct((_B, _N * _K), jnp.int32),
    mesh=_sc_mesh,
    compiler_params=pltpu.CompilerParams(needs_layout_passes=False),
    scratch_types=[
        pltpu.VMEM((_N,), jnp.float32),
        pltpu.VMEM((_N,), jnp.float32),
        pltpu.VMEM((_N * _K,), jnp.int32),
        pltpu.VMEM((_L * _CAP,), jnp.int32),
    ],
)(_sc_knn_body)


def kernel(coords, demands, capacity, W_amp, b_amp, W1, b1, W2, b2):
    xs = coords[:, :, 0]
    ys = coords[:, :, 1]
    c, s, nd, dd = _run_tc(xs, ys, demands, capacity, W1, b1, W2, b2)
    knn_flat = _sc_knn(xs, ys)
    isd = jnp.zeros((_B, _N), jnp.float32).at[:, 0].set(1.0)
    features = jnp.stack([xs, ys, nd, dd, isd], axis=-1)
    p = features @ W_amp + b_amp
    psix, psiy = _run_rot(p[..., 0], p[..., 1], c, s)
    psi_prime = jnp.stack([psix, psiy], axis=-1)
    knn = knn_flat.reshape(_B, _N, _K)
    return psi_prime, features, knn
